# Initial kernel scaffold; baseline (speedup 1.0000x reference)
import jax
import jax.numpy as jnp
from jax.experimental import pallas as pl


def kernel(input, ind, embed, fix):
    raise NotImplementedError("write your pallas kernel here")


# speedup vs baseline: 1.3425x; 1.3425x over previous
import functools

import jax
import jax.numpy as jnp
from jax import lax
from jax.experimental import pallas as pl
from jax.experimental.pallas import tpu as pltpu
from jax.experimental.pallas import tpu_sc as plsc

DIM = 32
N_EMBED = 8192
TOK = 8192
ROWS_PER_BLK = 256
NBLK = TOK // ROWS_PER_BLK

NC = 2
NS = 16
NW = NC * NS
BPW = TOK // NW
CH = 128
NCHUNK = BPW // CH
TPAD = 128


def _argmax_block(ind_ref, out_ref):
    x = ind_ref[...]
    m = jnp.max(x, axis=1, keepdims=True)
    col = lax.broadcasted_iota(jnp.int32, x.shape, 1)
    cand = jnp.where(x == m, col, N_EMBED)
    out_ref[0, 0, :] = jnp.min(cand, axis=1)


def _argmax_call(ind, interpret=False):
    out = pl.pallas_call(
        _argmax_block,
        grid=(NBLK,),
        in_specs=[pl.BlockSpec((ROWS_PER_BLK, N_EMBED), lambda i: (i, 0))],
        out_specs=pl.BlockSpec((1, 1, ROWS_PER_BLK), lambda i: (i, 0, 0)),
        out_shape=jax.ShapeDtypeStruct((NBLK, 1, ROWS_PER_BLK), jnp.int32),
        interpret=interpret,
    )(ind)
    return out.reshape(TOK)


def _sc_gather_body(tab_hbm, idx_hbm, inp_hbm, q_hbm, part_hbm,
                    idx_v, rows_v, inp_v, q_v, acc_v, sem):
    wid = lax.axis_index("s") * NC + lax.axis_index("c")
    acc = jnp.zeros((16,), jnp.float32)
    for t in range(NCHUNK):
        base = wid * BPW + t * CH
        pltpu.sync_copy(idx_hbm.at[pl.ds(base, CH)], idx_v)
        pltpu.async_copy(tab_hbm.at[idx_v], rows_v, sem).wait()
        pltpu.sync_copy(inp_hbm.at[pl.ds(base * DIM, CH * DIM)], inp_v)

        def body(r, a):
            v0 = rows_v[r, pl.ds(0, 16)]
            v1 = rows_v[r, pl.ds(16, 16)]
            q_v[pl.ds(r * DIM, 16)] = v0
            q_v[pl.ds(r * DIM + 16, 16)] = v1
            d0 = v0 - inp_v[pl.ds(r * DIM, 16)]
            d1 = v1 - inp_v[pl.ds(r * DIM + 16, 16)]
            return a + d0 * d0 + d1 * d1

        acc = lax.fori_loop(0, CH, body, acc)
        pltpu.sync_copy(q_v, q_hbm.at[pl.ds(base * DIM, CH * DIM)])
    acc_v[...] = acc
    pltpu.sync_copy(acc_v, part_hbm.at[pl.ds(wid * 16, 16)])


def _sc_gather(table, idx, flat_inp):
    k = functools.partial(
        pl.kernel,
        mesh=plsc.VectorSubcoreMesh(core_axis_name="c", subcore_axis_name="s"),
        out_type=[
            jax.ShapeDtypeStruct((TOK * DIM,), jnp.float32),
            jax.ShapeDtypeStruct((NW * 16,), jnp.float32),
        ],
        scratch_types=[
            pltpu.VMEM((CH,), jnp.int32),
            pltpu.VMEM((CH, TPAD), jnp.float32),
            pltpu.VMEM((CH * DIM,), jnp.float32),
            pltpu.VMEM((CH * DIM,), jnp.float32),
            pltpu.VMEM((16,), jnp.float32),
            pltpu.SemaphoreType.DMA,
        ],
    )(_sc_gather_body)
    return k(table, idx, flat_inp)


def kernel(input, ind, embed, fix):
    flatten = input.reshape(TOK * DIM)
    embed_ind = _argmax_call(ind)
    table = jnp.zeros((N_EMBED, TPAD), jnp.float32).at[:, :DIM].set(embed.T)
    quantize, part = _sc_gather(table, embed_ind, flatten)
    diff = jnp.sum(part) / (TOK * DIM)
    return (quantize.reshape(input.shape), diff,
            embed_ind.reshape(input.shape[:-1]))


# speedup vs baseline: 1.3985x; 1.0418x over previous
import functools

import jax
import jax.numpy as jnp
from jax import lax
from jax.experimental import pallas as pl
from jax.experimental.pallas import tpu as pltpu
from jax.experimental.pallas import tpu_sc as plsc

DIM = 32
N_EMBED = 8192
TOK = 8192
ROWS_PER_BLK = 512
NBLK = TOK // ROWS_PER_BLK

NC = 2
NS = 16
NW = NC * NS
BPW = TOK // NW
CH = 128
NCHUNK = BPW // CH
TPAD = 128


def _argmax_block(ind_ref, out_ref):
    x = ind_ref[...]
    m = jnp.max(x, axis=1, keepdims=True)
    col = lax.broadcasted_iota(jnp.int32, x.shape, 1)
    cand = jnp.where(x == m, col, N_EMBED)
    out_ref[0, 0, :] = jnp.min(cand, axis=1)


def _argmax_call(ind, interpret=False):
    out = pl.pallas_call(
        _argmax_block,
        grid=(NBLK,),
        in_specs=[pl.BlockSpec((ROWS_PER_BLK, N_EMBED), lambda i: (i, 0))],
        out_specs=pl.BlockSpec((1, 1, ROWS_PER_BLK), lambda i: (i, 0, 0)),
        out_shape=jax.ShapeDtypeStruct((NBLK, 1, ROWS_PER_BLK), jnp.int32),
        interpret=interpret,
    )(ind)
    return out.reshape(TOK)


def _sc_gather_body(tab_hbm, idx_hbm, inp_hbm, q_hbm, part_hbm,
                    idx_v, rows_v, inp_v, q_v, acc_v, sem):
    wid = lax.axis_index("s") * NC + lax.axis_index("c")
    acc = jnp.zeros((16,), jnp.float32)
    for t in range(NCHUNK):
        base = wid * BPW + t * CH
        pltpu.sync_copy(idx_hbm.at[pl.ds(base, CH)], idx_v)
        pltpu.async_copy(tab_hbm.at[idx_v], rows_v, sem).wait()
        pltpu.sync_copy(inp_hbm.at[pl.ds(base * DIM, CH * DIM)], inp_v)

        def body(r, a):
            v0 = rows_v[r, pl.ds(0, 16)]
            v1 = rows_v[r, pl.ds(16, 16)]
            q_v[pl.ds(r * DIM, 16)] = v0
            q_v[pl.ds(r * DIM + 16, 16)] = v1
            d0 = v0 - inp_v[pl.ds(r * DIM, 16)]
            d1 = v1 - inp_v[pl.ds(r * DIM + 16, 16)]
            return a + d0 * d0 + d1 * d1

        acc = lax.fori_loop(0, CH, body, acc)
        pltpu.sync_copy(q_v, q_hbm.at[pl.ds(base * DIM, CH * DIM)])
    acc_v[...] = acc
    pltpu.sync_copy(acc_v, part_hbm.at[pl.ds(wid * 16, 16)])


def _sc_gather(table, idx, flat_inp):
    k = functools.partial(
        pl.kernel,
        mesh=plsc.VectorSubcoreMesh(core_axis_name="c", subcore_axis_name="s"),
        out_type=[
            jax.ShapeDtypeStruct((TOK * DIM,), jnp.float32),
            jax.ShapeDtypeStruct((NW * 16,), jnp.float32),
        ],
        scratch_types=[
            pltpu.VMEM((CH,), jnp.int32),
            pltpu.VMEM((CH, TPAD), jnp.float32),
            pltpu.VMEM((CH * DIM,), jnp.float32),
            pltpu.VMEM((CH * DIM,), jnp.float32),
            pltpu.VMEM((16,), jnp.float32),
            pltpu.SemaphoreType.DMA,
        ],
    )(_sc_gather_body)
    return k(table, idx, flat_inp)


def kernel(input, ind, embed, fix):
    flatten = input.reshape(TOK * DIM)
    embed_ind = _argmax_call(ind)
    table = jnp.zeros((N_EMBED, TPAD), jnp.float32).at[:, :DIM].set(embed.T)
    quantize, part = _sc_gather(table, embed_ind, flatten)
    diff = jnp.sum(part) / (TOK * DIM)
    return (quantize.reshape(input.shape), diff,
            embed_ind.reshape(input.shape[:-1]))
